# initial kernel scaffold (unmeasured)
import jax
import jax.numpy as jnp
from jax import lax
from jax.experimental import pallas as pl
from jax.experimental.pallas import tpu as pltpu

M = 4096
K = 8192
N = 4096

BM = 1024
BN = 1024
BK = 1024


def _mm_body(a_ref, b_ref, o_ref, acc_ref):
    k = pl.program_id(2)

    @pl.when(k == 0)
    def _():
        acc_ref[...] = jnp.zeros_like(acc_ref)

    a = a_ref[...].astype(jnp.bfloat16)
    b = b_ref[...].astype(jnp.bfloat16)
    acc_ref[...] += lax.dot_general(
        a, b, (((1,), (1,)), ((), ())), preferred_element_type=jnp.float32
    )

    @pl.when(k == pl.num_programs(2) - 1)
    def _():
        o_ref[...] = acc_ref[...].astype(jnp.bfloat16)


def _partial_matmul(dy, w):
    return pl.pallas_call(
        _mm_body,
        grid=(M // BM, N // BN, K // BK),
        in_specs=[
            pl.BlockSpec((BM, BK), lambda m, n, k: (m, k)),
            pl.BlockSpec((BN, BK), lambda m, n, k: (n, k)),
        ],
        out_specs=pl.BlockSpec((BM, BN), lambda m, n, k: (m, n)),
        out_shape=jax.ShapeDtypeStruct((M, N), jnp.bfloat16),
        scratch_shapes=[pltpu.VMEM((BM, BN), jnp.float32)],
    )(dy, w)


ROWS = 512
NCHUNK = M // ROWS


def _ar_body(p_ref, out_ref, recv_ref, va_ref, vo_ref, send_sem, recv_sem,
             load_sem, store_sem):
    my_x = lax.axis_index("x")
    my_y = lax.axis_index("y")
    my_z = lax.axis_index("z")

    rdma = pltpu.make_async_remote_copy(
        src_ref=p_ref,
        dst_ref=recv_ref,
        send_sem=send_sem,
        recv_sem=recv_sem,
        device_id=(1 - my_x, my_y, my_z),
        device_id_type=pl.DeviceIdType.MESH,
    )
    rdma.start()
    rdma.wait()

    for i in range(NCHUNK):
        sl = pl.ds(i * ROWS, ROWS)
        ld = pltpu.make_async_copy(p_ref.at[sl, :], va_ref, load_sem)
        ld.start()
        ld.wait()
        vo_ref[...] = va_ref[...].astype(jnp.float32) + recv_ref[sl, :].astype(
            jnp.float32
        )
        st = pltpu.make_async_copy(vo_ref, out_ref.at[sl, :], store_sem)
        st.start()
        st.wait()


def _exchange_add(p):
    return pl.pallas_call(
        _ar_body,
        out_shape=jax.ShapeDtypeStruct((M, N), jnp.float32),
        in_specs=[pl.BlockSpec(memory_space=pltpu.MemorySpace.ANY)],
        out_specs=pl.BlockSpec(memory_space=pltpu.MemorySpace.ANY),
        scratch_shapes=[
            pltpu.VMEM((M, N), jnp.bfloat16),
            pltpu.VMEM((ROWS, N), jnp.bfloat16),
            pltpu.VMEM((ROWS, N), jnp.float32),
            pltpu.SemaphoreType.DMA,
            pltpu.SemaphoreType.DMA,
            pltpu.SemaphoreType.DMA,
            pltpu.SemaphoreType.DMA,
        ],
        compiler_params=pltpu.CompilerParams(collective_id=0),
    )(p)


def kernel(dy, W):
    p = _partial_matmul(dy, W)
    return _exchange_add(p)


# baseline (device time: 865374 ns/iter reference)
import jax
import jax.numpy as jnp
from jax import lax
from jax.experimental import pallas as pl
from jax.experimental.pallas import tpu as pltpu

M = 4096
K = 8192
N = 4096

BM = 1024
BN = 1024
BK = 1024


def _mm_body(a_ref, b_ref, o_ref, acc_ref):
    k = pl.program_id(2)

    @pl.when(k == 0)
    def _():
        acc_ref[...] = jnp.zeros_like(acc_ref)

    a = a_ref[...].astype(jnp.bfloat16)
    b = b_ref[...].astype(jnp.bfloat16)
    acc_ref[...] += lax.dot_general(
        a, b, (((1,), (1,)), ((), ())), preferred_element_type=jnp.float32
    )

    @pl.when(k == pl.num_programs(2) - 1)
    def _():
        o_ref[...] = acc_ref[...].astype(jnp.bfloat16)


def _partial_matmul(dy, w):
    return pl.pallas_call(
        _mm_body,
        grid=(M // BM, N // BN, K // BK),
        in_specs=[
            pl.BlockSpec((BM, BK), lambda m, n, k: (m, k)),
            pl.BlockSpec((BN, BK), lambda m, n, k: (n, k)),
        ],
        out_specs=pl.BlockSpec((BM, BN), lambda m, n, k: (m, n)),
        out_shape=jax.ShapeDtypeStruct((M, N), jnp.bfloat16),
        scratch_shapes=[pltpu.VMEM((BM, BN), jnp.float32)],
    )(dy, w)


ROWS = 512
NCHUNK = M // ROWS


def _ar_body(p_ref, out_ref, recv_ref, va_ref, vo_ref, send_sem, recv_sem,
             load_sem, store_sem):
    my_x = lax.axis_index("x")
    my_y = lax.axis_index("y")
    my_z = lax.axis_index("z")

    rdma = pltpu.make_async_remote_copy(
        src_ref=p_ref,
        dst_ref=recv_ref,
        send_sem=send_sem,
        recv_sem=recv_sem,
        device_id=(1 - my_x, my_y, my_z),
        device_id_type=pl.DeviceIdType.MESH,
    )
    rdma.start()
    rdma.wait()

    for i in range(NCHUNK):
        sl = pl.ds(i * ROWS, ROWS)
        ld = pltpu.make_async_copy(p_ref.at[sl, :], va_ref, load_sem)
        ld.start()
        ld.wait()
        vo_ref[...] = va_ref[...].astype(jnp.float32) + recv_ref[sl, :].astype(
            jnp.float32
        )
        st = pltpu.make_async_copy(vo_ref, out_ref.at[sl, :], store_sem)
        st.start()
        st.wait()


def _exchange_add(p):
    return pl.pallas_call(
        _ar_body,
        out_shape=jax.ShapeDtypeStruct((M, N), jnp.float32),
        in_specs=[pl.BlockSpec(memory_space=pl.ANY)],
        out_specs=pl.BlockSpec(memory_space=pl.ANY),
        scratch_shapes=[
            pltpu.VMEM((M, N), jnp.bfloat16),
            pltpu.VMEM((ROWS, N), jnp.bfloat16),
            pltpu.VMEM((ROWS, N), jnp.float32),
            pltpu.SemaphoreType.DMA,
            pltpu.SemaphoreType.DMA,
            pltpu.SemaphoreType.DMA,
            pltpu.SemaphoreType.DMA,
        ],
        compiler_params=pltpu.CompilerParams(
            vmem_limit_bytes=60 * 1024 * 1024,
        ),
    )(p)


def kernel(dy, W):
    p = _partial_matmul(dy, W)
    return _exchange_add(p)


# device time: 706183 ns/iter; 1.2254x vs baseline; 1.2254x over previous
import jax
import jax.numpy as jnp
from jax import lax
from jax.experimental import pallas as pl
from jax.experimental.pallas import tpu as pltpu

M = 4096
K = 8192
N = 4096

BM = 1024
BN = 1024
BK = 1024


def _mm_body(a_ref, b_ref, o_ref, acc_ref):
    k = pl.program_id(2)

    @pl.when(k == 0)
    def _():
        acc_ref[...] = jnp.zeros_like(acc_ref)

    a = a_ref[...].astype(jnp.bfloat16)
    b = b_ref[...].astype(jnp.bfloat16)
    acc_ref[...] += lax.dot_general(
        a, b, (((1,), (1,)), ((), ())), preferred_element_type=jnp.float32
    )

    @pl.when(k == pl.num_programs(2) - 1)
    def _():
        o_ref[...] = acc_ref[...].astype(jnp.bfloat16)


def _partial_matmul(dy, w):
    return pl.pallas_call(
        _mm_body,
        grid=(M // BM, N // BN, K // BK),
        in_specs=[
            pl.BlockSpec((BM, BK), lambda m, n, k: (m, k)),
            pl.BlockSpec((BN, BK), lambda m, n, k: (n, k)),
        ],
        out_specs=pl.BlockSpec((BM, BN), lambda m, n, k: (m, n)),
        out_shape=jax.ShapeDtypeStruct((M, N), jnp.bfloat16),
        scratch_shapes=[pltpu.VMEM((BM, BN), jnp.float32)],
    )(dy, w)


Q = M // 4
H = Q // 2
ROWS = 512


def _ar_body(p_ref, out_ref, rall_ref, rx_ref, va_ref, vo_ref,
             send_sems, recv_sems, load_sem, store_sem):
    my_x = lax.axis_index("x")
    my_y = lax.axis_index("y")
    my_z = lax.axis_index("z")

    def store_quarter(slot):
        for i in range(Q // ROWS):
            sl = pl.ds(slot * Q + i * ROWS, ROWS)
            vo_ref[...] = rall_ref[sl, :].astype(jnp.float32)
            st = pltpu.make_async_copy(vo_ref, out_ref.at[sl, :], store_sem)
            st.start()
            st.wait()

    for yy in range(2):
        for zz in range(2):

            @pl.when((my_y == yy) & (my_z == zz))
            def _(yy=yy, zz=zz):
                q = 2 * yy + zz
                q_z = 2 * yy + (1 - zz)
                q_y = 2 * (1 - yy) + zz
                q_d = 2 * (1 - yy) + (1 - zz)

                x_dev = (1 - my_x, my_y, my_z)
                z_dev = (my_x, my_y, 1 - my_z)
                y_dev = (my_x, 1 - my_y, my_z)

                x_rdma = pltpu.make_async_remote_copy(
                    src_ref=p_ref.at[pl.ds(q * Q, Q), :],
                    dst_ref=rx_ref,
                    send_sem=send_sems.at[0],
                    recv_sem=recv_sems.at[0],
                    device_id=x_dev,
                    device_id_type=pl.DeviceIdType.MESH,
                )
                x_rdma.start()
                x_rdma.wait()

                for i in range(Q // ROWS):
                    rsl = pl.ds(i * ROWS, ROWS)
                    gsl = pl.ds(q * Q + i * ROWS, ROWS)
                    ld = pltpu.make_async_copy(
                        p_ref.at[gsl, :], va_ref, load_sem
                    )
                    ld.start()
                    ld.wait()
                    s32 = va_ref[...].astype(jnp.float32) + rx_ref[
                        rsl, :
                    ].astype(jnp.float32)
                    rall_ref[gsl, :] = s32.astype(jnp.bfloat16)
                    vo_ref[...] = s32
                    st = pltpu.make_async_copy(
                        vo_ref, out_ref.at[gsl, :], store_sem
                    )
                    st.start()
                    st.wait()

                qsl = pl.ds(q * Q, Q)
                a_z = pltpu.make_async_remote_copy(
                    src_ref=rall_ref.at[qsl, :], dst_ref=rall_ref.at[qsl, :],
                    send_sem=send_sems.at[1], recv_sem=recv_sems.at[1],
                    device_id=z_dev, device_id_type=pl.DeviceIdType.MESH,
                )
                a_y = pltpu.make_async_remote_copy(
                    src_ref=rall_ref.at[qsl, :], dst_ref=rall_ref.at[qsl, :],
                    send_sem=send_sems.at[2], recv_sem=recv_sems.at[2],
                    device_id=y_dev, device_id_type=pl.DeviceIdType.MESH,
                )
                a_z.start()
                a_y.start()
                a_z.wait()
                a_y.wait()

                b_z = pltpu.make_async_remote_copy(
                    src_ref=rall_ref.at[pl.ds(q_y * Q, H), :],
                    dst_ref=rall_ref.at[pl.ds(q_y * Q, H), :],
                    send_sem=send_sems.at[3], recv_sem=recv_sems.at[3],
                    device_id=z_dev, device_id_type=pl.DeviceIdType.MESH,
                )
                b_y = pltpu.make_async_remote_copy(
                    src_ref=rall_ref.at[pl.ds(q_z * Q + H, H), :],
                    dst_ref=rall_ref.at[pl.ds(q_z * Q + H, H), :],
                    send_sem=send_sems.at[4], recv_sem=recv_sems.at[4],
                    device_id=y_dev, device_id_type=pl.DeviceIdType.MESH,
                )
                b_z.start()
                b_y.start()

                store_quarter(q_z)
                store_quarter(q_y)

                b_z.wait()
                b_y.wait()
                store_quarter(q_d)


def _exchange_add(p):
    return pl.pallas_call(
        _ar_body,
        out_shape=jax.ShapeDtypeStruct((M, N), jnp.float32),
        in_specs=[pl.BlockSpec(memory_space=pl.ANY)],
        out_specs=pl.BlockSpec(memory_space=pl.ANY),
        scratch_shapes=[
            pltpu.VMEM((M, N), jnp.bfloat16),
            pltpu.VMEM((Q, N), jnp.bfloat16),
            pltpu.VMEM((ROWS, N), jnp.bfloat16),
            pltpu.VMEM((ROWS, N), jnp.float32),
            pltpu.SemaphoreType.DMA((5,)),
            pltpu.SemaphoreType.DMA((5,)),
            pltpu.SemaphoreType.DMA,
            pltpu.SemaphoreType.DMA,
        ],
        compiler_params=pltpu.CompilerParams(
            vmem_limit_bytes=62 * 1024 * 1024,
        ),
    )(p)


def kernel(dy, W):
    p = _partial_matmul(dy, W)
    return _exchange_add(p)


# device time: 375966 ns/iter; 2.3017x vs baseline; 1.8783x over previous
import jax
import jax.numpy as jnp
from jax import lax
from jax.experimental import pallas as pl
from jax.experimental.pallas import tpu as pltpu

M = 4096
K = 8192
N = 4096

MQ = M // 4
BM = 1024
BN = 1024
BK = 1024


def _mm_body(a_ref, b_ref, o_ref, acc_ref):
    k = pl.program_id(2)

    @pl.when(k == 0)
    def _():
        acc_ref[...] = jnp.zeros_like(acc_ref)

    a = a_ref[...].astype(jnp.bfloat16)
    b = b_ref[...].astype(jnp.bfloat16)
    acc_ref[...] += lax.dot_general(
        a, b, (((1,), (1,)), ((), ())), preferred_element_type=jnp.float32
    )

    @pl.when(k == pl.num_programs(2) - 1)
    def _():
        o_ref[...] = acc_ref[...].astype(jnp.bfloat16)


def _partial_matmul(dy_q, w):
    return pl.pallas_call(
        _mm_body,
        grid=(MQ // BM, N // BN, K // BK),
        in_specs=[
            pl.BlockSpec((BM, BK), lambda m, n, k: (m, k)),
            pl.BlockSpec((BN, BK), lambda m, n, k: (n, k)),
        ],
        out_specs=pl.BlockSpec((BM, BN), lambda m, n, k: (m, n)),
        out_shape=jax.ShapeDtypeStruct((MQ, N), jnp.bfloat16),
        scratch_shapes=[pltpu.VMEM((BM, BN), jnp.float32)],
    )(dy_q, w)


Q = M // 4
H = Q // 2
ROWS = 512


def _ar_body_v4(p_ref, out_ref, rall_ref, rx_ref, va_ref, vo_ref,
                send_sems, recv_sems, load_sem, store_sem):
    my_x = lax.axis_index("x")
    my_y = lax.axis_index("y")
    my_z = lax.axis_index("z")

    def store_quarter(slot):
        for i in range(Q // ROWS):
            sl = pl.ds(slot * Q + i * ROWS, ROWS)
            vo_ref[...] = rall_ref[sl, :].astype(jnp.float32)
            st = pltpu.make_async_copy(vo_ref, out_ref.at[sl, :], store_sem)
            st.start()
            st.wait()

    for yy in range(2):
        for zz in range(2):

            @pl.when((my_y == yy) & (my_z == zz))
            def _(yy=yy, zz=zz):
                q = 2 * yy + zz
                q_z = 2 * yy + (1 - zz)
                q_y = 2 * (1 - yy) + zz
                q_d = 2 * (1 - yy) + (1 - zz)

                x_dev = (1 - my_x, my_y, my_z)
                z_dev = (my_x, my_y, 1 - my_z)
                y_dev = (my_x, 1 - my_y, my_z)

                def remote(src_sl, dst_sl, sem_i, dev):
                    return pltpu.make_async_remote_copy(
                        src_ref=rall_ref.at[src_sl, :],
                        dst_ref=rall_ref.at[dst_sl, :],
                        send_sem=send_sems.at[sem_i],
                        recv_sem=recv_sems.at[sem_i],
                        device_id=dev,
                        device_id_type=pl.DeviceIdType.MESH,
                    )

                ld = pltpu.make_async_copy(p_ref, va_ref, load_sem)
                ld.start()

                x_rdma = [
                    pltpu.make_async_remote_copy(
                        src_ref=p_ref.at[pl.ds(h * H, H), :],
                        dst_ref=rx_ref.at[pl.ds(h * H, H), :],
                        send_sem=send_sems.at[h],
                        recv_sem=recv_sems.at[h],
                        device_id=x_dev,
                        device_id_type=pl.DeviceIdType.MESH,
                    )
                    for h in range(2)
                ]
                x_rdma[0].start()
                x_rdma[1].start()
                ld.wait()

                a_ops = []
                st_prev = None
                for h in range(2):
                    x_rdma[h].wait()
                    if st_prev is not None:
                        st_prev.wait()
                    hsl = pl.ds(h * H, H)
                    gsl = pl.ds(q * Q + h * H, H)
                    vo_ref[...] = va_ref[hsl, :].astype(
                        jnp.float32
                    ) + rx_ref[hsl, :].astype(jnp.float32)
                    rall_ref[gsl, :] = vo_ref[...].astype(jnp.bfloat16)
                    a_z = remote(gsl, gsl, 2 + 2 * h, z_dev)
                    a_y = remote(gsl, gsl, 3 + 2 * h, y_dev)
                    a_z.start()
                    a_y.start()
                    a_ops.append((a_z, a_y))
                    st_prev = pltpu.make_async_copy(
                        vo_ref, out_ref.at[gsl, :], store_sem
                    )
                    st_prev.start()
                st_prev.wait()

                a_ops[0][1].wait()
                b_z = remote(pl.ds(q_y * Q, H), pl.ds(q_y * Q, H), 6, z_dev)
                b_z.start()
                a_ops[1][0].wait()
                b_y = remote(
                    pl.ds(q_z * Q + H, H), pl.ds(q_z * Q + H, H), 7, y_dev
                )
                b_y.start()

                a_ops[0][0].wait()
                a_ops[1][1].wait()
                store_quarter(q_z)
                store_quarter(q_y)

                b_z.wait()
                b_y.wait()
                store_quarter(q_d)


def _ar_body(p_ref, out_ref, rall_ref, rx_ref, va_ref, vo_ref,
             send_sems, recv_sems, load_sem, store_sem):
    my_x = lax.axis_index("x")
    my_y = lax.axis_index("y")
    my_z = lax.axis_index("z")

    def store_quarter(slot):
        for i in range(Q // ROWS):
            sl = pl.ds(slot * Q + i * ROWS, ROWS)
            vo_ref[...] = rall_ref[sl, :].astype(jnp.float32)
            st = pltpu.make_async_copy(vo_ref, out_ref.at[sl, :], store_sem)
            st.start()
            st.wait()

    for yy in range(2):
        for zz in range(2):

            @pl.when((my_y == yy) & (my_z == zz))
            def _(yy=yy, zz=zz):
                q = 2 * yy + zz
                q_z = 2 * yy + (1 - zz)
                q_y = 2 * (1 - yy) + zz
                q_d = 2 * (1 - yy) + (1 - zz)

                x_dev = (1 - my_x, my_y, my_z)
                z_dev = (my_x, my_y, 1 - my_z)
                y_dev = (my_x, 1 - my_y, my_z)

                x_rdma = pltpu.make_async_remote_copy(
                    src_ref=p_ref,
                    dst_ref=rx_ref,
                    send_sem=send_sems.at[0],
                    recv_sem=recv_sems.at[0],
                    device_id=x_dev,
                    device_id_type=pl.DeviceIdType.MESH,
                )
                x_rdma.start()
                x_rdma.wait()

                for i in range(Q // ROWS):
                    rsl = pl.ds(i * ROWS, ROWS)
                    gsl = pl.ds(q * Q + i * ROWS, ROWS)
                    ld = pltpu.make_async_copy(
                        p_ref.at[rsl, :], va_ref, load_sem
                    )
                    ld.start()
                    ld.wait()
                    s32 = va_ref[...].astype(jnp.float32) + rx_ref[
                        rsl, :
                    ].astype(jnp.float32)
                    rall_ref[gsl, :] = s32.astype(jnp.bfloat16)
                    vo_ref[...] = s32
                    st = pltpu.make_async_copy(
                        vo_ref, out_ref.at[gsl, :], store_sem
                    )
                    st.start()
                    st.wait()

                qsl = pl.ds(q * Q, Q)
                a_z = pltpu.make_async_remote_copy(
                    src_ref=rall_ref.at[qsl, :], dst_ref=rall_ref.at[qsl, :],
                    send_sem=send_sems.at[1], recv_sem=recv_sems.at[1],
                    device_id=z_dev, device_id_type=pl.DeviceIdType.MESH,
                )
                a_y = pltpu.make_async_remote_copy(
                    src_ref=rall_ref.at[qsl, :], dst_ref=rall_ref.at[qsl, :],
                    send_sem=send_sems.at[2], recv_sem=recv_sems.at[2],
                    device_id=y_dev, device_id_type=pl.DeviceIdType.MESH,
                )
                a_z.start()
                a_y.start()
                a_z.wait()
                a_y.wait()

                b_z = pltpu.make_async_remote_copy(
                    src_ref=rall_ref.at[pl.ds(q_y * Q, H), :],
                    dst_ref=rall_ref.at[pl.ds(q_y * Q, H), :],
                    send_sem=send_sems.at[3], recv_sem=recv_sems.at[3],
                    device_id=z_dev, device_id_type=pl.DeviceIdType.MESH,
                )
                b_y = pltpu.make_async_remote_copy(
                    src_ref=rall_ref.at[pl.ds(q_z * Q + H, H), :],
                    dst_ref=rall_ref.at[pl.ds(q_z * Q + H, H), :],
                    send_sem=send_sems.at[4], recv_sem=recv_sems.at[4],
                    device_id=y_dev, device_id_type=pl.DeviceIdType.MESH,
                )
                b_z.start()
                b_y.start()

                store_quarter(q_z)
                store_quarter(q_y)

                b_z.wait()
                b_y.wait()
                store_quarter(q_d)


def _exchange_add(p):
    return pl.pallas_call(
        _ar_body_v4,
        out_shape=jax.ShapeDtypeStruct((M, N), jnp.float32),
        in_specs=[pl.BlockSpec(memory_space=pl.ANY)],
        out_specs=pl.BlockSpec(memory_space=pl.ANY),
        scratch_shapes=[
            pltpu.VMEM((M, N), jnp.bfloat16),
            pltpu.VMEM((Q, N), jnp.bfloat16),
            pltpu.VMEM((Q, N), jnp.bfloat16),
            pltpu.VMEM((ROWS, N), jnp.float32),
            pltpu.SemaphoreType.DMA((8,)),
            pltpu.SemaphoreType.DMA((8,)),
            pltpu.SemaphoreType.DMA,
            pltpu.SemaphoreType.DMA,
        ],
        compiler_params=pltpu.CompilerParams(
            vmem_limit_bytes=62 * 1024 * 1024,
        ),
    )(p)


def kernel(dy, W):
    q = 2 * lax.axis_index("y") + lax.axis_index("z")
    dy_q = lax.dynamic_slice_in_dim(dy, q * MQ, MQ, axis=0)
    p = _partial_matmul(dy_q, W)
    return _exchange_add(p)


# device time: 351332 ns/iter; 2.4631x vs baseline; 1.0701x over previous
import jax
import jax.numpy as jnp
from jax import lax
from jax.experimental import pallas as pl
from jax.experimental.pallas import tpu as pltpu

M = 4096
K = 8192
N = 4096

MQ = M // 4
BM = 1024
BN = 1024
BK = 1024


def _mm_body(a_ref, b_ref, o_ref, acc_ref):
    k = pl.program_id(2)

    @pl.when(k == 0)
    def _():
        acc_ref[...] = jnp.zeros_like(acc_ref)

    a = a_ref[...].astype(jnp.bfloat16)
    b = b_ref[...].astype(jnp.bfloat16)
    acc_ref[...] += lax.dot_general(
        a, b, (((1,), (1,)), ((), ())), preferred_element_type=jnp.float32
    )

    @pl.when(k == pl.num_programs(2) - 1)
    def _():
        o_ref[...] = acc_ref[...].astype(jnp.bfloat16)


def _partial_matmul(dy, w):
    qblocks = MQ // BM

    def dy_index(m, n, k):
        q = 2 * lax.axis_index("y") + lax.axis_index("z")
        return (q * qblocks + m, k)

    return pl.pallas_call(
        _mm_body,
        grid=(MQ // BM, N // BN, K // BK),
        in_specs=[
            pl.BlockSpec((BM, BK), dy_index),
            pl.BlockSpec((BN, BK), lambda m, n, k: (n, k)),
        ],
        out_specs=pl.BlockSpec((BM, BN), lambda m, n, k: (m, n)),
        out_shape=jax.ShapeDtypeStruct((MQ, N), jnp.bfloat16),
        scratch_shapes=[pltpu.VMEM((BM, BN), jnp.float32)],
    )(dy, w)


Q = M // 4
H = Q // 2
ROWS = 512


def _ar_body_v4(p_ref, out_ref, rall_ref, rx_ref, va_ref, vo_ref,
                send_sems, recv_sems, load_sem, store_sem):
    my_x = lax.axis_index("x")
    my_y = lax.axis_index("y")
    my_z = lax.axis_index("z")

    def store_quarter(slot):
        for i in range(Q // ROWS):
            sl = pl.ds(slot * Q + i * ROWS, ROWS)
            vo_ref[...] = rall_ref[sl, :].astype(jnp.float32)
            st = pltpu.make_async_copy(vo_ref, out_ref.at[sl, :], store_sem)
            st.start()
            st.wait()

    for yy in range(2):
        for zz in range(2):

            @pl.when((my_y == yy) & (my_z == zz))
            def _(yy=yy, zz=zz):
                q = 2 * yy + zz
                q_z = 2 * yy + (1 - zz)
                q_y = 2 * (1 - yy) + zz
                q_d = 2 * (1 - yy) + (1 - zz)

                x_dev = (1 - my_x, my_y, my_z)
                z_dev = (my_x, my_y, 1 - my_z)
                y_dev = (my_x, 1 - my_y, my_z)

                def remote(src_sl, dst_sl, sem_i, dev):
                    return pltpu.make_async_remote_copy(
                        src_ref=rall_ref.at[src_sl, :],
                        dst_ref=rall_ref.at[dst_sl, :],
                        send_sem=send_sems.at[sem_i],
                        recv_sem=recv_sems.at[sem_i],
                        device_id=dev,
                        device_id_type=pl.DeviceIdType.MESH,
                    )

                ld = pltpu.make_async_copy(p_ref, va_ref, load_sem)
                ld.start()

                x_rdma = [
                    pltpu.make_async_remote_copy(
                        src_ref=p_ref.at[pl.ds(h * H, H), :],
                        dst_ref=rx_ref.at[pl.ds(h * H, H), :],
                        send_sem=send_sems.at[h],
                        recv_sem=recv_sems.at[h],
                        device_id=x_dev,
                        device_id_type=pl.DeviceIdType.MESH,
                    )
                    for h in range(2)
                ]
                x_rdma[0].start()
                x_rdma[1].start()
                ld.wait()

                a_ops = []
                st_prev = None
                for h in range(2):
                    x_rdma[h].wait()
                    if st_prev is not None:
                        st_prev.wait()
                    hsl = pl.ds(h * H, H)
                    gsl = pl.ds(q * Q + h * H, H)
                    vo_ref[...] = va_ref[hsl, :].astype(
                        jnp.float32
                    ) + rx_ref[hsl, :].astype(jnp.float32)
                    rall_ref[gsl, :] = vo_ref[...].astype(jnp.bfloat16)
                    a_z = remote(gsl, gsl, 2 + 2 * h, z_dev)
                    a_y = remote(gsl, gsl, 3 + 2 * h, y_dev)
                    a_z.start()
                    a_y.start()
                    a_ops.append((a_z, a_y))
                    st_prev = pltpu.make_async_copy(
                        vo_ref, out_ref.at[gsl, :], store_sem
                    )
                    st_prev.start()
                st_prev.wait()

                a_ops[0][1].wait()
                b_z = remote(pl.ds(q_y * Q, H), pl.ds(q_y * Q, H), 6, z_dev)
                b_z.start()
                a_ops[1][0].wait()
                b_y = remote(
                    pl.ds(q_z * Q + H, H), pl.ds(q_z * Q + H, H), 7, y_dev
                )
                b_y.start()

                a_ops[0][0].wait()
                a_ops[1][1].wait()
                store_quarter(q_z)
                store_quarter(q_y)

                b_z.wait()
                b_y.wait()
                store_quarter(q_d)


def _ar_body(p_ref, out_ref, rall_ref, rx_ref, va_ref, vo_ref,
             send_sems, recv_sems, load_sem, store_sem):
    my_x = lax.axis_index("x")
    my_y = lax.axis_index("y")
    my_z = lax.axis_index("z")

    def store_quarter(slot):
        for i in range(Q // ROWS):
            sl = pl.ds(slot * Q + i * ROWS, ROWS)
            vo_ref[...] = rall_ref[sl, :].astype(jnp.float32)
            st = pltpu.make_async_copy(vo_ref, out_ref.at[sl, :], store_sem)
            st.start()
            st.wait()

    for yy in range(2):
        for zz in range(2):

            @pl.when((my_y == yy) & (my_z == zz))
            def _(yy=yy, zz=zz):
                q = 2 * yy + zz
                q_z = 2 * yy + (1 - zz)
                q_y = 2 * (1 - yy) + zz
                q_d = 2 * (1 - yy) + (1 - zz)

                x_dev = (1 - my_x, my_y, my_z)
                z_dev = (my_x, my_y, 1 - my_z)
                y_dev = (my_x, 1 - my_y, my_z)

                x_rdma = pltpu.make_async_remote_copy(
                    src_ref=p_ref,
                    dst_ref=rx_ref,
                    send_sem=send_sems.at[0],
                    recv_sem=recv_sems.at[0],
                    device_id=x_dev,
                    device_id_type=pl.DeviceIdType.MESH,
                )
                x_rdma.start()
                x_rdma.wait()

                for i in range(Q // ROWS):
                    rsl = pl.ds(i * ROWS, ROWS)
                    gsl = pl.ds(q * Q + i * ROWS, ROWS)
                    ld = pltpu.make_async_copy(
                        p_ref.at[rsl, :], va_ref, load_sem
                    )
                    ld.start()
                    ld.wait()
                    s32 = va_ref[...].astype(jnp.float32) + rx_ref[
                        rsl, :
                    ].astype(jnp.float32)
                    rall_ref[gsl, :] = s32.astype(jnp.bfloat16)
                    vo_ref[...] = s32
                    st = pltpu.make_async_copy(
                        vo_ref, out_ref.at[gsl, :], store_sem
                    )
                    st.start()
                    st.wait()

                qsl = pl.ds(q * Q, Q)
                a_z = pltpu.make_async_remote_copy(
                    src_ref=rall_ref.at[qsl, :], dst_ref=rall_ref.at[qsl, :],
                    send_sem=send_sems.at[1], recv_sem=recv_sems.at[1],
                    device_id=z_dev, device_id_type=pl.DeviceIdType.MESH,
                )
                a_y = pltpu.make_async_remote_copy(
                    src_ref=rall_ref.at[qsl, :], dst_ref=rall_ref.at[qsl, :],
                    send_sem=send_sems.at[2], recv_sem=recv_sems.at[2],
                    device_id=y_dev, device_id_type=pl.DeviceIdType.MESH,
                )
                a_z.start()
                a_y.start()
                a_z.wait()
                a_y.wait()

                b_z = pltpu.make_async_remote_copy(
                    src_ref=rall_ref.at[pl.ds(q_y * Q, H), :],
                    dst_ref=rall_ref.at[pl.ds(q_y * Q, H), :],
                    send_sem=send_sems.at[3], recv_sem=recv_sems.at[3],
                    device_id=z_dev, device_id_type=pl.DeviceIdType.MESH,
                )
                b_y = pltpu.make_async_remote_copy(
                    src_ref=rall_ref.at[pl.ds(q_z * Q + H, H), :],
                    dst_ref=rall_ref.at[pl.ds(q_z * Q + H, H), :],
                    send_sem=send_sems.at[4], recv_sem=recv_sems.at[4],
                    device_id=y_dev, device_id_type=pl.DeviceIdType.MESH,
                )
                b_z.start()
                b_y.start()

                store_quarter(q_z)
                store_quarter(q_y)

                b_z.wait()
                b_y.wait()
                store_quarter(q_d)


def _exchange_add(p):
    return pl.pallas_call(
        _ar_body_v4,
        out_shape=jax.ShapeDtypeStruct((M, N), jnp.float32),
        in_specs=[pl.BlockSpec(memory_space=pl.ANY)],
        out_specs=pl.BlockSpec(memory_space=pl.ANY),
        scratch_shapes=[
            pltpu.VMEM((M, N), jnp.bfloat16),
            pltpu.VMEM((Q, N), jnp.bfloat16),
            pltpu.VMEM((Q, N), jnp.bfloat16),
            pltpu.VMEM((ROWS, N), jnp.float32),
            pltpu.SemaphoreType.DMA((8,)),
            pltpu.SemaphoreType.DMA((8,)),
            pltpu.SemaphoreType.DMA,
            pltpu.SemaphoreType.DMA,
        ],
        compiler_params=pltpu.CompilerParams(
            vmem_limit_bytes=62 * 1024 * 1024,
        ),
    )(p)


def kernel(dy, W):
    p = _partial_matmul(dy, W)
    return _exchange_add(p)


# device time: 339123 ns/iter; 2.5518x vs baseline; 1.0360x over previous
import jax
import jax.numpy as jnp
from jax import lax
from jax.experimental import pallas as pl
from jax.experimental.pallas import tpu as pltpu

M = 4096
K = 8192
N = 4096

MQ = M // 4
BM = 1024
BN = 1024
BK = 1024


def _mm_body(a_ref, b_ref, o_ref, acc_ref):
    k = pl.program_id(2)

    @pl.when(k == 0)
    def _():
        acc_ref[...] = jnp.zeros_like(acc_ref)

    a = a_ref[...].astype(jnp.bfloat16)
    b = b_ref[...].astype(jnp.bfloat16)
    acc_ref[...] += lax.dot_general(
        a, b, (((1,), (1,)), ((), ())), preferred_element_type=jnp.float32
    )

    @pl.when(k == pl.num_programs(2) - 1)
    def _():
        o_ref[...] = acc_ref[...].astype(jnp.bfloat16)


def _mmx_body(dy_ref, w_ref, p_ref, rx_ref, p_vmem, acc_ref,
              send_sems, recv_sems, cp_sem):
    n = pl.program_id(1)
    k = pl.program_id(2)
    n_last = pl.num_programs(1) - 1
    k_last = pl.num_programs(2) - 1

    @pl.when(k == 0)
    def _():
        acc_ref[...] = jnp.zeros_like(acc_ref)

    a = dy_ref[...].astype(jnp.bfloat16)
    b = w_ref[...].astype(jnp.bfloat16)
    acc_ref[...] += lax.dot_general(
        a, b, (((1,), (1,)), ((), ())), preferred_element_type=jnp.float32
    )

    my_x = lax.axis_index("x")
    my_y = lax.axis_index("y")
    my_z = lax.axis_index("z")
    x_dev = (1 - my_x, my_y, my_z)

    @pl.when(k == k_last)
    def _():
        for j in range(N // BN):

            @pl.when(n == j)
            def _(j=j):
                csl = pl.ds(j * BN, BN)
                p_vmem[:, csl] = acc_ref[...].astype(jnp.bfloat16)
                xs = pltpu.make_async_remote_copy(
                    src_ref=p_vmem.at[:, csl],
                    dst_ref=rx_ref.at[:, csl],
                    send_sem=send_sems.at[j],
                    recv_sem=recv_sems.at[j],
                    device_id=x_dev,
                    device_id_type=pl.DeviceIdType.MESH,
                )
                xs.start()

    @pl.when((n == n_last) & (k == k_last))
    def _():
        for j in range(N // BN):
            xw = pltpu.make_async_remote_copy(
                src_ref=p_vmem.at[:, pl.ds(j * BN, BN)],
                dst_ref=rx_ref.at[:, pl.ds(j * BN, BN)],
                send_sem=send_sems.at[j],
                recv_sem=recv_sems.at[j],
                device_id=x_dev,
                device_id_type=pl.DeviceIdType.MESH,
            )
            xw.wait()
        cp = pltpu.make_async_copy(p_vmem, p_ref, cp_sem)
        cp.start()
        cp.wait()


def _partial_matmul(dy, w):
    qblocks = MQ // BM

    def dy_index(m, n, k):
        q = 2 * lax.axis_index("y") + lax.axis_index("z")
        return (q * qblocks + m, k)

    return pl.pallas_call(
        _mmx_body,
        grid=(MQ // BM, N // BN, K // BK),
        in_specs=[
            pl.BlockSpec((BM, BK), dy_index),
            pl.BlockSpec((BN, BK), lambda m, n, k: (n, k)),
        ],
        out_specs=[
            pl.BlockSpec(memory_space=pl.ANY),
            pl.BlockSpec(memory_space=pl.ANY),
        ],
        out_shape=[
            jax.ShapeDtypeStruct((MQ, N), jnp.bfloat16),
            jax.ShapeDtypeStruct((MQ, N), jnp.bfloat16),
        ],
        scratch_shapes=[
            pltpu.VMEM((MQ, N), jnp.bfloat16),
            pltpu.VMEM((BM, BN), jnp.float32),
            pltpu.SemaphoreType.DMA((4,)),
            pltpu.SemaphoreType.DMA((4,)),
            pltpu.SemaphoreType.DMA,
        ],
        compiler_params=pltpu.CompilerParams(
            vmem_limit_bytes=48 * 1024 * 1024,
        ),
    )(dy, w)


Q = M // 4
H = Q // 2
ROWS = 512


def _ar_body_v6(p_ref, rx_ref, out_ref, rall_ref, va_ref, vb_ref, vo_ref,
                send_sems, recv_sems, load_sem, load2_sem, store_sem):
    my_x = lax.axis_index("x")
    my_y = lax.axis_index("y")
    my_z = lax.axis_index("z")

    def store_quarter(slot):
        for i in range(Q // ROWS):
            sl = pl.ds(slot * Q + i * ROWS, ROWS)
            vo_ref[...] = rall_ref[sl, :].astype(jnp.float32)
            st = pltpu.make_async_copy(vo_ref, out_ref.at[sl, :], store_sem)
            st.start()
            st.wait()

    for yy in range(2):
        for zz in range(2):

            @pl.when((my_y == yy) & (my_z == zz))
            def _(yy=yy, zz=zz):
                q = 2 * yy + zz
                q_z = 2 * yy + (1 - zz)
                q_y = 2 * (1 - yy) + zz
                q_d = 2 * (1 - yy) + (1 - zz)

                z_dev = (my_x, my_y, 1 - my_z)
                y_dev = (my_x, 1 - my_y, my_z)

                def remote(src_sl, dst_sl, sem_i, dev):
                    return pltpu.make_async_remote_copy(
                        src_ref=rall_ref.at[src_sl, :],
                        dst_ref=rall_ref.at[dst_sl, :],
                        send_sem=send_sems.at[sem_i],
                        recv_sem=recv_sems.at[sem_i],
                        device_id=dev,
                        device_id_type=pl.DeviceIdType.MESH,
                    )

                ld = pltpu.make_async_copy(p_ref, va_ref, load_sem)
                ld2 = pltpu.make_async_copy(rx_ref, vb_ref, load2_sem)
                ld.start()
                ld2.start()
                ld.wait()
                ld2.wait()

                a_ops = []
                st_prev = None
                for h in range(2):
                    if st_prev is not None:
                        st_prev.wait()
                    hsl = pl.ds(h * H, H)
                    gsl = pl.ds(q * Q + h * H, H)
                    vo_ref[...] = va_ref[hsl, :].astype(
                        jnp.float32
                    ) + vb_ref[hsl, :].astype(jnp.float32)
                    rall_ref[gsl, :] = vo_ref[...].astype(jnp.bfloat16)
                    a_z = remote(gsl, gsl, 2 + 2 * h, z_dev)
                    a_y = remote(gsl, gsl, 3 + 2 * h, y_dev)
                    a_z.start()
                    a_y.start()
                    a_ops.append((a_z, a_y))
                    st_prev = pltpu.make_async_copy(
                        vo_ref, out_ref.at[gsl, :], store_sem
                    )
                    st_prev.start()
                st_prev.wait()

                a_ops[0][1].wait()
                b_z = remote(pl.ds(q_y * Q, H), pl.ds(q_y * Q, H), 6, z_dev)
                b_z.start()
                a_ops[1][0].wait()
                b_y = remote(
                    pl.ds(q_z * Q + H, H), pl.ds(q_z * Q + H, H), 7, y_dev
                )
                b_y.start()

                a_ops[0][0].wait()
                a_ops[1][1].wait()
                store_quarter(q_z)
                store_quarter(q_y)

                b_z.wait()
                b_y.wait()
                store_quarter(q_d)


def _ar_body(p_ref, out_ref, rall_ref, rx_ref, va_ref, vo_ref,
             send_sems, recv_sems, load_sem, store_sem):
    my_x = lax.axis_index("x")
    my_y = lax.axis_index("y")
    my_z = lax.axis_index("z")

    def store_quarter(slot):
        for i in range(Q // ROWS):
            sl = pl.ds(slot * Q + i * ROWS, ROWS)
            vo_ref[...] = rall_ref[sl, :].astype(jnp.float32)
            st = pltpu.make_async_copy(vo_ref, out_ref.at[sl, :], store_sem)
            st.start()
            st.wait()

    for yy in range(2):
        for zz in range(2):

            @pl.when((my_y == yy) & (my_z == zz))
            def _(yy=yy, zz=zz):
                q = 2 * yy + zz
                q_z = 2 * yy + (1 - zz)
                q_y = 2 * (1 - yy) + zz
                q_d = 2 * (1 - yy) + (1 - zz)

                x_dev = (1 - my_x, my_y, my_z)
                z_dev = (my_x, my_y, 1 - my_z)
                y_dev = (my_x, 1 - my_y, my_z)

                x_rdma = pltpu.make_async_remote_copy(
                    src_ref=p_ref,
                    dst_ref=rx_ref,
                    send_sem=send_sems.at[0],
                    recv_sem=recv_sems.at[0],
                    device_id=x_dev,
                    device_id_type=pl.DeviceIdType.MESH,
                )
                x_rdma.start()
                x_rdma.wait()

                for i in range(Q // ROWS):
                    rsl = pl.ds(i * ROWS, ROWS)
                    gsl = pl.ds(q * Q + i * ROWS, ROWS)
                    ld = pltpu.make_async_copy(
                        p_ref.at[rsl, :], va_ref, load_sem
                    )
                    ld.start()
                    ld.wait()
                    s32 = va_ref[...].astype(jnp.float32) + rx_ref[
                        rsl, :
                    ].astype(jnp.float32)
                    rall_ref[gsl, :] = s32.astype(jnp.bfloat16)
                    vo_ref[...] = s32
                    st = pltpu.make_async_copy(
                        vo_ref, out_ref.at[gsl, :], store_sem
                    )
                    st.start()
                    st.wait()

                qsl = pl.ds(q * Q, Q)
                a_z = pltpu.make_async_remote_copy(
                    src_ref=rall_ref.at[qsl, :], dst_ref=rall_ref.at[qsl, :],
                    send_sem=send_sems.at[1], recv_sem=recv_sems.at[1],
                    device_id=z_dev, device_id_type=pl.DeviceIdType.MESH,
                )
                a_y = pltpu.make_async_remote_copy(
                    src_ref=rall_ref.at[qsl, :], dst_ref=rall_ref.at[qsl, :],
                    send_sem=send_sems.at[2], recv_sem=recv_sems.at[2],
                    device_id=y_dev, device_id_type=pl.DeviceIdType.MESH,
                )
                a_z.start()
                a_y.start()
                a_z.wait()
                a_y.wait()

                b_z = pltpu.make_async_remote_copy(
                    src_ref=rall_ref.at[pl.ds(q_y * Q, H), :],
                    dst_ref=rall_ref.at[pl.ds(q_y * Q, H), :],
                    send_sem=send_sems.at[3], recv_sem=recv_sems.at[3],
                    device_id=z_dev, device_id_type=pl.DeviceIdType.MESH,
                )
                b_y = pltpu.make_async_remote_copy(
                    src_ref=rall_ref.at[pl.ds(q_z * Q + H, H), :],
                    dst_ref=rall_ref.at[pl.ds(q_z * Q + H, H), :],
                    send_sem=send_sems.at[4], recv_sem=recv_sems.at[4],
                    device_id=y_dev, device_id_type=pl.DeviceIdType.MESH,
                )
                b_z.start()
                b_y.start()

                store_quarter(q_z)
                store_quarter(q_y)

                b_z.wait()
                b_y.wait()
                store_quarter(q_d)


def _exchange_add(p, rx):
    return pl.pallas_call(
        _ar_body_v6,
        out_shape=jax.ShapeDtypeStruct((M, N), jnp.float32),
        in_specs=[
            pl.BlockSpec(memory_space=pl.ANY),
            pl.BlockSpec(memory_space=pl.ANY),
        ],
        out_specs=pl.BlockSpec(memory_space=pl.ANY),
        scratch_shapes=[
            pltpu.VMEM((M, N), jnp.bfloat16),
            pltpu.VMEM((Q, N), jnp.bfloat16),
            pltpu.VMEM((Q, N), jnp.bfloat16),
            pltpu.VMEM((ROWS, N), jnp.float32),
            pltpu.SemaphoreType.DMA((8,)),
            pltpu.SemaphoreType.DMA((8,)),
            pltpu.SemaphoreType.DMA,
            pltpu.SemaphoreType.DMA,
            pltpu.SemaphoreType.DMA,
        ],
        compiler_params=pltpu.CompilerParams(
            vmem_limit_bytes=62 * 1024 * 1024,
        ),
    )(p, rx)


def kernel(dy, W):
    p, rx = _partial_matmul(dy, W)
    return _exchange_add(p, rx)
